# baseline (device time: 53338 ns/iter reference)
import jax
import jax.numpy as jnp
from jax import lax
from jax.experimental import pallas as pl
from jax.experimental.pallas import tpu as pltpu

N_DEV = 4
SEQ = 1024
S_PER = 256
D = 1024
N_HEADS = 8
DH = 128
SCALE = 0.08838834764831843

FROM_LEFT, FROM_RIGHT, FROM_DIAG = 0, 1, 2
SPLIT = 2
SP_ROWS = S_PER // SPLIT


def kernel(x, Wq, Wo, Wk, Wv):
    def body(x_ref, wq_ref, wo_ref, wk_ref, wv_ref, out_ref,
             xg_ref, ag_send_sems, ag_recv_sems,
             wqb_ref, wkb_ref, wvb_ref, wob_ref,
             q_ref, k_ref, v_ref, attn_t_ref,
             rs_send, rs_recv, rs_send_sems, rs_recv_sems):
        my_pos = lax.axis_index("i")
        left = (my_pos - 1) % N_DEV
        right = (my_pos + 1) % N_DEV
        diag = (my_pos + 2) % N_DEV

        barrier_sem = pltpu.get_barrier_semaphore()
        for nbr in [left, right, diag]:
            pl.semaphore_signal(
                barrier_sem, inc=1,
                device_id=(nbr,), device_id_type=pl.DeviceIdType.MESH,
            )
        pl.semaphore_wait(barrier_sem, 3)

        def block(pos):
            return pl.ds(pos * S_PER, S_PER)

        def subrows(pos, part):
            return pl.ds(pos * S_PER + part * SP_ROWS, SP_ROWS)

        ag_sends = []
        for slot, tgt in ((FROM_LEFT, right), (FROM_RIGHT, left),
                          (FROM_DIAG, diag)):
            for part in range(SPLIT):
                rdma = pltpu.make_async_remote_copy(
                    src_ref=x_ref.at[0, pl.ds(part * SP_ROWS, SP_ROWS), :],
                    dst_ref=xg_ref.at[subrows(my_pos, part), :],
                    send_sem=ag_send_sems.at[slot * SPLIT + part],
                    recv_sem=ag_recv_sems.at[slot * SPLIT + part],
                    device_id=(tgt,),
                    device_id_type=pl.DeviceIdType.MESH,
                )
                rdma.start()
                ag_sends.append(rdma)

        wqb_ref[:, :] = (wq_ref[:, :] * SCALE).astype(jnp.bfloat16)
        wkb_ref[:, :] = wk_ref[:, :].astype(jnp.bfloat16)
        wvb_ref[:, :] = wv_ref[:, :].astype(jnp.bfloat16)
        wob_ref[:, :] = wo_ref[:, :].astype(jnp.bfloat16)
        xg_ref[block(my_pos), :] = x_ref[0, :, :]

        def qkv_rows(rows):
            xc = xg_ref[rows, :]
            q_ref[rows, :] = jnp.dot(
                xc, wqb_ref[:, :],
                preferred_element_type=jnp.float32).astype(jnp.bfloat16)
            k_ref[rows, :] = jnp.dot(
                xc, wkb_ref[:, :],
                preferred_element_type=jnp.float32).astype(jnp.bfloat16)
            v_ref[rows, :] = jnp.dot(
                xc, wvb_ref[:, :],
                preferred_element_type=jnp.float32).astype(jnp.bfloat16)

        qkv_rows(block(my_pos))

        for part in range(SPLIT):
            for slot, origin in ((FROM_LEFT, left), (FROM_RIGHT, right),
                                 (FROM_DIAG, diag)):
                recv = pltpu.make_async_remote_copy(
                    src_ref=xg_ref.at[subrows(origin, part), :],
                    dst_ref=xg_ref.at[subrows(origin, part), :],
                    send_sem=ag_send_sems.at[slot * SPLIT + part],
                    recv_sem=ag_recv_sems.at[slot * SPLIT + part],
                    device_id=(origin,),
                    device_id_type=pl.DeviceIdType.MESH,
                )
                recv.wait_recv()
                qkv_rows(subrows(origin, part))

        for rdma in ag_sends:
            rdma.wait_send()

        def attn_block(pos):
            for h in range(N_HEADS):
                sl = pl.ds(h * DH, DH)
                qh = q_ref[block(pos), sl]
                kh = k_ref[:, sl]
                vh = v_ref[:, sl]
                s_t = lax.dot_general(
                    kh, qh, (((1,), (1,)), ((), ())),
                    preferred_element_type=jnp.float32).astype(jnp.bfloat16)
                e_t = jnp.exp(s_t)
                linv = 1.0 / jnp.sum(e_t, axis=0, keepdims=True,
                                     dtype=jnp.float32)
                oh_t = lax.dot_general(
                    vh, e_t, (((0,), (0,)), ((), ())),
                    preferred_element_type=jnp.float32)
                attn_t_ref[sl, :] = (oh_t * linv).astype(jnp.bfloat16)

        rs_sends = []
        for slot, tgt in ((FROM_LEFT, right), (FROM_RIGHT, left),
                          (FROM_DIAG, diag)):
            attn_block(tgt)
            rs_send[slot, :, :] = lax.dot_general(
                attn_t_ref[:, :], wob_ref[:, :], (((0,), (0,)), ((), ())),
                preferred_element_type=jnp.float32).astype(jnp.bfloat16)
            rdma = pltpu.make_async_remote_copy(
                src_ref=rs_send.at[slot],
                dst_ref=rs_recv.at[slot],
                send_sem=rs_send_sems.at[slot],
                recv_sem=rs_recv_sems.at[slot],
                device_id=(tgt,),
                device_id_type=pl.DeviceIdType.MESH,
            )
            rdma.start()
            rs_sends.append(rdma)

        attn_block(my_pos)
        acc = lax.dot_general(
            attn_t_ref[:, :], wob_ref[:, :], (((0,), (0,)), ((), ())),
            preferred_element_type=jnp.float32)

        for slot, origin in ((FROM_LEFT, left), (FROM_RIGHT, right),
                             (FROM_DIAG, diag)):
            recv = pltpu.make_async_remote_copy(
                src_ref=rs_send.at[slot],
                dst_ref=rs_recv.at[slot],
                send_sem=rs_send_sems.at[slot],
                recv_sem=rs_recv_sems.at[slot],
                device_id=(origin,),
                device_id_type=pl.DeviceIdType.MESH,
            )
            recv.wait_recv()
            acc = acc + rs_recv[slot, :, :].astype(jnp.float32)

        out_ref[0, :, :] = acc

        for rdma in rs_sends:
            rdma.wait_send()

    xb = x.astype(jnp.bfloat16)

    return pl.pallas_call(
        body,
        out_shape=jax.ShapeDtypeStruct((1, S_PER, D), jnp.float32),
        in_specs=[pl.BlockSpec(memory_space=pltpu.VMEM)] * 5,
        out_specs=pl.BlockSpec(memory_space=pltpu.VMEM),
        scratch_shapes=[
            pltpu.VMEM((SEQ, D), jnp.bfloat16),
            pltpu.SemaphoreType.DMA((3 * SPLIT,)),
            pltpu.SemaphoreType.DMA((3 * SPLIT,)),
            pltpu.VMEM((D, D), jnp.bfloat16),
            pltpu.VMEM((D, D), jnp.bfloat16),
            pltpu.VMEM((D, D), jnp.bfloat16),
            pltpu.VMEM((D, D), jnp.bfloat16),
            pltpu.VMEM((SEQ, D), jnp.bfloat16),
            pltpu.VMEM((SEQ, D), jnp.bfloat16),
            pltpu.VMEM((SEQ, D), jnp.bfloat16),
            pltpu.VMEM((D, S_PER), jnp.bfloat16),
            pltpu.VMEM((3, S_PER, D), jnp.bfloat16),
            pltpu.VMEM((3, S_PER, D), jnp.bfloat16),
            pltpu.SemaphoreType.DMA((3,)),
            pltpu.SemaphoreType.DMA((3,)),
        ],
        compiler_params=pltpu.CompilerParams(
            collective_id=0, vmem_limit_bytes=60 * 1024 * 1024,
        ),
    )(xb, Wq, Wo, Wk, Wv)


# device time: 45984 ns/iter; 1.1599x vs baseline; 1.1599x over previous
import jax
import jax.numpy as jnp
from jax import lax
from jax.experimental import pallas as pl
from jax.experimental.pallas import tpu as pltpu

N_DEV = 4
SEQ = 1024
S_PER = 256
D = 1024
N_HEADS = 8
DH = 128
SCALE = 0.08838834764831843

FROM_LEFT, FROM_RIGHT, FROM_DIAG = 0, 1, 2
SPLIT = 2
SP_ROWS = S_PER // SPLIT


def kernel(x, Wq, Wo, Wk, Wv):
    def body(x_ref, wq_ref, wo_ref, wk_ref, wv_ref, out_ref,
             xg_ref, ag_send_sems, ag_recv_sems,
             wqb_ref, wkb_ref, wvb_ref, wob_ref,
             q_ref, k_ref, v_ref, attn_ref,
             rs_send, rs_recv, rs_send_sems, rs_recv_sems):
        my_pos = lax.axis_index("i")
        left = (my_pos - 1) % N_DEV
        right = (my_pos + 1) % N_DEV
        diag = (my_pos + 2) % N_DEV

        barrier_sem = pltpu.get_barrier_semaphore()
        for nbr in [left, right, diag]:
            pl.semaphore_signal(
                barrier_sem, inc=1,
                device_id=(nbr,), device_id_type=pl.DeviceIdType.MESH,
            )
        pl.semaphore_wait(barrier_sem, 3)

        def block(pos):
            return pl.ds(pos * S_PER, S_PER)

        def subrows(pos, part):
            return pl.ds(pos * S_PER + part * SP_ROWS, SP_ROWS)

        ag_sends = []
        for slot, tgt in ((FROM_LEFT, right), (FROM_RIGHT, left),
                          (FROM_DIAG, diag)):
            for part in range(SPLIT):
                rdma = pltpu.make_async_remote_copy(
                    src_ref=x_ref.at[0, pl.ds(part * SP_ROWS, SP_ROWS), :],
                    dst_ref=xg_ref.at[subrows(my_pos, part), :],
                    send_sem=ag_send_sems.at[slot * SPLIT + part],
                    recv_sem=ag_recv_sems.at[slot * SPLIT + part],
                    device_id=(tgt,),
                    device_id_type=pl.DeviceIdType.MESH,
                )
                rdma.start()
                ag_sends.append(rdma)

        wqb_ref[:, :] = (wq_ref[:, :] * SCALE).astype(jnp.bfloat16)
        wkb_ref[:, :] = wk_ref[:, :].astype(jnp.bfloat16)
        wvb_ref[:, :] = wv_ref[:, :].astype(jnp.bfloat16)
        wob_ref[:, :] = wo_ref[:, :].astype(jnp.bfloat16)
        xg_ref[block(my_pos), :] = x_ref[0, :, :]

        def qkv_rows(rows):
            xc = xg_ref[rows, :]
            q_ref[rows, :] = jnp.dot(
                xc, wqb_ref[:, :],
                preferred_element_type=jnp.float32).astype(jnp.bfloat16)
            k_ref[rows, :] = jnp.dot(
                xc, wkb_ref[:, :],
                preferred_element_type=jnp.float32).astype(jnp.bfloat16)
            v_ref[rows, :] = jnp.dot(
                xc, wvb_ref[:, :],
                preferred_element_type=jnp.float32).astype(jnp.bfloat16)

        qkv_rows(block(my_pos))

        for part in range(SPLIT):
            for slot, origin in ((FROM_LEFT, left), (FROM_RIGHT, right),
                                 (FROM_DIAG, diag)):
                recv = pltpu.make_async_remote_copy(
                    src_ref=xg_ref.at[subrows(origin, part), :],
                    dst_ref=xg_ref.at[subrows(origin, part), :],
                    send_sem=ag_send_sems.at[slot * SPLIT + part],
                    recv_sem=ag_recv_sems.at[slot * SPLIT + part],
                    device_id=(origin,),
                    device_id_type=pl.DeviceIdType.MESH,
                )
                recv.wait_recv()
                qkv_rows(subrows(origin, part))

        for rdma in ag_sends:
            rdma.wait_send()

        def attn_block(pos):
            for h in range(N_HEADS):
                sl = pl.ds(h * DH, DH)
                qh = q_ref[block(pos), sl]
                kh = k_ref[:, sl]
                vh = v_ref[:, sl]
                s = lax.dot_general(
                    qh, kh, (((1,), (1,)), ((), ())),
                    preferred_element_type=jnp.float32).astype(jnp.bfloat16)
                e = jnp.exp(s)
                linv = 1.0 / jnp.sum(e, axis=1, keepdims=True,
                                     dtype=jnp.float32)
                oh = jnp.dot(e, vh, preferred_element_type=jnp.float32)
                attn_ref[block(pos), sl] = (oh * linv).astype(jnp.bfloat16)

        rs_sends = []
        for slot, tgt in ((FROM_DIAG, diag), (FROM_LEFT, right),
                          (FROM_RIGHT, left)):
            attn_block(tgt)
            rs_send[slot, :, :] = jnp.dot(
                attn_ref[block(tgt), :], wob_ref[:, :],
                preferred_element_type=jnp.float32).astype(jnp.bfloat16)
            rdma = pltpu.make_async_remote_copy(
                src_ref=rs_send.at[slot],
                dst_ref=rs_recv.at[slot],
                send_sem=rs_send_sems.at[slot],
                recv_sem=rs_recv_sems.at[slot],
                device_id=(tgt,),
                device_id_type=pl.DeviceIdType.MESH,
            )
            rdma.start()
            rs_sends.append(rdma)

        attn_block(my_pos)
        acc = jnp.dot(attn_ref[block(my_pos), :], wob_ref[:, :],
                      preferred_element_type=jnp.float32)

        for slot, origin in ((FROM_LEFT, left), (FROM_RIGHT, right),
                             (FROM_DIAG, diag)):
            recv = pltpu.make_async_remote_copy(
                src_ref=rs_send.at[slot],
                dst_ref=rs_recv.at[slot],
                send_sem=rs_send_sems.at[slot],
                recv_sem=rs_recv_sems.at[slot],
                device_id=(origin,),
                device_id_type=pl.DeviceIdType.MESH,
            )
            recv.wait_recv()
            acc = acc + rs_recv[slot, :, :].astype(jnp.float32)

        out_ref[0, :, :] = acc

        for rdma in rs_sends:
            rdma.wait_send()

    xb = x.astype(jnp.bfloat16)

    return pl.pallas_call(
        body,
        out_shape=jax.ShapeDtypeStruct((1, S_PER, D), jnp.float32),
        in_specs=[pl.BlockSpec(memory_space=pltpu.VMEM)] * 5,
        out_specs=pl.BlockSpec(memory_space=pltpu.VMEM),
        scratch_shapes=[
            pltpu.VMEM((SEQ, D), jnp.bfloat16),
            pltpu.SemaphoreType.DMA((3 * SPLIT,)),
            pltpu.SemaphoreType.DMA((3 * SPLIT,)),
            pltpu.VMEM((D, D), jnp.bfloat16),
            pltpu.VMEM((D, D), jnp.bfloat16),
            pltpu.VMEM((D, D), jnp.bfloat16),
            pltpu.VMEM((D, D), jnp.bfloat16),
            pltpu.VMEM((SEQ, D), jnp.bfloat16),
            pltpu.VMEM((SEQ, D), jnp.bfloat16),
            pltpu.VMEM((SEQ, D), jnp.bfloat16),
            pltpu.VMEM((SEQ, D), jnp.bfloat16),
            pltpu.VMEM((3, S_PER, D), jnp.bfloat16),
            pltpu.VMEM((3, S_PER, D), jnp.bfloat16),
            pltpu.SemaphoreType.DMA((3,)),
            pltpu.SemaphoreType.DMA((3,)),
        ],
        compiler_params=pltpu.CompilerParams(
            collective_id=0, vmem_limit_bytes=60 * 1024 * 1024,
        ),
    )(xb, Wq, Wo, Wk, Wv)
